# baseline (speedup 1.0000x reference)
import jax
import jax.numpy as jnp
from jax.experimental import pallas as pl

_NUM_CLASSES = 80
_K = 300
_LANES = 128


def _topk_kernel(x_ref, scores_ref, idx_ref):
    x = x_ref[0]
    R, L = x.shape
    row_iota = jax.lax.broadcasted_iota(jnp.int32, (R, L), 0)
    col_iota = jax.lax.broadcasted_iota(jnp.int32, (R, L), 1)
    flat_iota = row_iota * L + col_iota
    k_iota = jax.lax.broadcasted_iota(jnp.int32, (_K, L), 0)
    big = jnp.int32(2**31 - 1)
    neg = jnp.float32(-jnp.inf)

    def body(i, carry):
        x, vals, idxs = carry
        m = jnp.max(x)
        amin = jnp.min(jnp.where(x == m, flat_iota, big))
        sel = k_iota == i
        vals = jnp.where(sel, m, vals)
        idxs = jnp.where(sel, amin, idxs)
        x = jnp.where(flat_iota == amin, neg, x)
        return x, vals, idxs

    init = (
        x,
        jnp.zeros((_K, L), jnp.float32),
        jnp.zeros((_K, L), jnp.int32),
    )
    _, vals, idxs = jax.lax.fori_loop(0, _K, body, init)
    scores_ref[0] = jax.nn.sigmoid(vals)
    idx_ref[0] = idxs


def kernel(pred_logits, pred_boxes, orig_target_sizes):
    B, Q, C = pred_logits.shape
    QC = Q * C
    R = QC // _LANES
    flat = pred_logits.reshape(B, R, _LANES)
    scores_t, idx_t = pl.pallas_call(
        _topk_kernel,
        grid=(B,),
        in_specs=[pl.BlockSpec((1, R, _LANES), lambda b: (b, 0, 0))],
        out_specs=[
            pl.BlockSpec((1, _K, _LANES), lambda b: (b, 0, 0)),
            pl.BlockSpec((1, _K, _LANES), lambda b: (b, 0, 0)),
        ],
        out_shape=[
            jax.ShapeDtypeStruct((B, _K, _LANES), jnp.float32),
            jax.ShapeDtypeStruct((B, _K, _LANES), jnp.int32),
        ],
    )(flat)
    scores = scores_t[:, :, 0]
    index_flat = idx_t[:, :, 0]

    labels = index_flat % C
    query_indices = index_flat // C

    top_raw = jnp.take_along_axis(pred_boxes, query_indices[:, :, None], axis=1)
    cx, cy, w, h = jnp.split(top_raw, 4, axis=-1)
    xyxy = jnp.concatenate(
        [cx - 0.5 * w, cy - 0.5 * h, cx + 0.5 * w, cy + 0.5 * h], axis=-1
    )
    img_wh = orig_target_sizes[:, None, :]
    scale_fct = jnp.concatenate([img_wh, img_wh], axis=2)
    top_boxes = xyxy * scale_fct
    return scores, labels, top_boxes
